# Initial kernel scaffold; baseline (speedup 1.0000x reference)
import jax
import jax.numpy as jnp
from jax.experimental import pallas as pl


def kernel(weights, starts, ends, factor_depth_coords, ray_indices, num_rays):
    raise NotImplementedError("write your pallas kernel here")


# speedup vs baseline: 23.9520x; 23.9520x over previous
import functools

import jax
import jax.numpy as jnp
from jax import lax
from jax.experimental import pallas as pl
from jax.experimental.pallas import tpu as pltpu
from jax.experimental.pallas import tpu_sc as plsc

NUM_S = 4194304
NUM_R = 65536
NC = 2
NS = 16
NW = NC * NS
LANE = 128
ROWS = NUM_S // LANE
ROWS_PER_W = ROWS // NW
CHUNK_ROWS = 128
N_CHUNKS = ROWS_PER_W // CHUNK_ROWS

PREP_BLK = 1024
PREP_GRID = ROWS // PREP_BLK


def _prep_body(w_ref, s_ref, e_ref, val_ref, mn_ref, mx_ref):
    steps = (s_ref[...] + e_ref[...]) * 0.5
    val_ref[...] = w_ref[...] * steps
    mn_ref[...] = jnp.full((1, 1, LANE), jnp.min(steps), jnp.float32)
    mx_ref[...] = jnp.full((1, 1, LANE), jnp.max(steps), jnp.float32)


def _tc_prep(w, s, e):
    blk = pl.BlockSpec((PREP_BLK, LANE), lambda g: (g, 0))
    row = pl.BlockSpec((1, 1, LANE), lambda g: (g, 0, 0))
    return pl.pallas_call(
        _prep_body,
        grid=(PREP_GRID,),
        in_specs=[blk, blk, blk],
        out_specs=[blk, row, row],
        out_shape=[
            jax.ShapeDtypeStruct((ROWS, LANE), jnp.float32),
            jax.ShapeDtypeStruct((PREP_GRID, 1, LANE), jnp.float32),
            jax.ShapeDtypeStruct((PREP_GRID, 1, LANE), jnp.float32),
        ],
    )(w, s, e)


def _sc_body(val_hbm, w_hbm, idx_hbm, tabs_hbm,
             valb, wb, idxb, zb, dtab, atab, sem):
    c = lax.axis_index("c")
    s = lax.axis_index("s")
    wid = c * NS + s

    stripe = NUM_R // NS

    def _zero(i, _):
        zb[pl.ds(i * 16, 16)] = jnp.zeros((16,), jnp.float32)
        return 0

    lax.fori_loop(0, stripe // 16, _zero, 0)
    pltpu.sync_copy(zb, dtab.at[pl.ds(s * stripe, stripe)])
    pltpu.sync_copy(zb, atab.at[pl.ds(s * stripe, stripe)])
    plsc.subcore_barrier()

    row0 = wid * ROWS_PER_W

    def _chunk(ck, _):
        rb = row0 + ck * CHUNK_ROWS
        pltpu.sync_copy(val_hbm.at[pl.ds(rb, CHUNK_ROWS)], valb)
        pltpu.sync_copy(w_hbm.at[pl.ds(rb, CHUNK_ROWS)], wb)
        pltpu.sync_copy(idx_hbm.at[pl.ds(rb, CHUNK_ROWS)], idxb)

        def _fire(j, _):
            pltpu.async_copy(valb.at[j], dtab.at[idxb.at[j]], sem, add=True)
            pltpu.async_copy(wb.at[j], atab.at[idxb.at[j]], sem, add=True)
            return 0

        lax.fori_loop(0, CHUNK_ROWS, _fire, 0)

        def _drain(j, _):
            pltpu.make_async_copy(valb.at[0], dtab.at[idxb.at[0]], sem).wait()
            pltpu.make_async_copy(wb.at[0], atab.at[idxb.at[0]], sem).wait()
            return 0

        lax.fori_loop(0, CHUNK_ROWS, _drain, 0)
        return 0

    lax.fori_loop(0, N_CHUNKS, _chunk, 0)
    plsc.subcore_barrier()

    @pl.when(s == 0)
    def _dump():
        pltpu.sync_copy(dtab, tabs_hbm.at[c, 0])
        pltpu.sync_copy(atab, tabs_hbm.at[c, 1])


def _sc_scatter(val, w, idx):
    mesh = plsc.VectorSubcoreMesh(core_axis_name="c", subcore_axis_name="s")
    return pl.kernel(
        _sc_body,
        out_type=jax.ShapeDtypeStruct((NC, 2, NUM_R), jnp.float32),
        mesh=mesh,
        scratch_types=[
            pltpu.VMEM((CHUNK_ROWS, LANE), jnp.float32),
            pltpu.VMEM((CHUNK_ROWS, LANE), jnp.float32),
            pltpu.VMEM((CHUNK_ROWS, LANE), jnp.int32),
            pltpu.VMEM((NUM_R // NS,), jnp.float32),
            pltpu.VMEM_SHARED((NUM_R,), jnp.float32),
            pltpu.VMEM_SHARED((NUM_R,), jnp.float32),
            pltpu.SemaphoreType.DMA,
        ],
    )(val, w, idx)


def _finish_body(d0_ref, d1_ref, a0_ref, a1_ref, mn_ref, mx_ref, fac_ref,
                 out_ref):
    depth = (d0_ref[...] + d1_ref[...]) / (a0_ref[...] + a1_ref[...] + 1e-10)
    mn = jnp.min(mn_ref[...])
    mx = jnp.max(mx_ref[...])
    out_ref[...] = jnp.clip(depth, mn, mx) * fac_ref[...]


def _tc_finish(d0, d1, a0, a1, mn, mx, fac):
    return pl.pallas_call(
        _finish_body,
        out_shape=jax.ShapeDtypeStruct((NUM_R // LANE, LANE), jnp.float32),
    )(d0, d1, a0, a1, mn, mx, fac)


def kernel(weights, starts, ends, factor_depth_coords, ray_indices, num_rays):
    del num_rays
    w2 = weights.reshape(ROWS, LANE)
    s2 = starts.reshape(ROWS, LANE)
    e2 = ends.reshape(ROWS, LANE)
    val, mn, mx = _tc_prep(w2, s2, e2)
    idx = ray_indices.astype(jnp.int32).reshape(ROWS, LANE)
    tabs = _sc_scatter(val, w2, idx)
    tr = NUM_R // LANE
    d0 = tabs[0, 0].reshape(tr, LANE)
    d1 = tabs[1, 0].reshape(tr, LANE)
    a0 = tabs[0, 1].reshape(tr, LANE)
    a1 = tabs[1, 1].reshape(tr, LANE)
    fac = factor_depth_coords.reshape(tr, LANE)
    out = _tc_finish(d0, d1, a0, a1, mn, mx, fac)
    return out.reshape(NUM_R, 1)
